# Initial kernel scaffold; baseline (speedup 1.0000x reference)
import jax
import jax.numpy as jnp
from jax.experimental import pallas as pl


def kernel(x, edge_index, edge_attr, initial_state, W1, b1, W2, b2, Wfc, bfc):
    raise NotImplementedError("write your pallas kernel here")


# speedup vs baseline: 13.2241x; 13.2241x over previous
import functools

import jax
import jax.numpy as jnp
from jax import lax
from jax.experimental import pallas as pl
from jax.experimental.pallas import tpu as pltpu
from jax.experimental.pallas import tpu_sc as plsc

N = 10000
E = 320000
HID = 128
OUT = 3
INIT_DIM = 8

NC = 2
NS = 16
NW = NC * NS
CHUNK = 128
NCHUNK = 80
EPW = CHUNK * NCHUNK
EP = NW * EPW
NPAD = 10240
RPT = NPAD // NS
DEGW = 16


def _make_sc_degree():
    mesh = plsc.VectorSubcoreMesh(core_axis_name="c", subcore_axis_name="s")

    @functools.partial(
        pl.kernel,
        out_type=jax.ShapeDtypeStruct((NC, NPAD, DEGW), jnp.float32),
        mesh=mesh,
        compiler_params=pltpu.CompilerParams(use_tc_tiling_on_sc=False),
        scratch_types=[
            pltpu.VMEM((NCHUNK, CHUNK), jnp.int32),
            pltpu.VMEM((CHUNK, DEGW), jnp.float32),
            pltpu.VMEM_SHARED((NPAD, DEGW), jnp.float32),
        ],
    )
    def deg_kernel(dst_hbm, z_hbm, out_hbm, didx, ones, acc):
        cid = lax.axis_index("c")
        sid = lax.axis_index("s")
        wid = sid * NC + cid
        pltpu.sync_copy(dst_hbm.at[wid], didx)

        def fill_ones(i, carry):
            ones[i, :] = jnp.full((DEGW,), 1.0, jnp.float32)
            return carry

        lax.fori_loop(0, CHUNK, fill_ones, 0)
        pltpu.sync_copy(z_hbm, acc.at[pl.ds(sid * RPT, RPT)])
        plsc.subcore_barrier()

        def body(j, carry):
            pltpu.sync_copy(ones, acc.at[didx.at[j]], add=True)
            return carry

        lax.fori_loop(0, NCHUNK, body, 0)
        plsc.subcore_barrier()
        pltpu.sync_copy(
            acc.at[pl.ds(sid * RPT, RPT)],
            out_hbm.at[cid, pl.ds(sid * RPT, RPT)],
        )

    return deg_kernel


def _make_sc_propagate(D):
    mesh = plsc.VectorSubcoreMesh(core_axis_name="c", subcore_axis_name="s")
    lanes_per_row = D // 16

    @functools.partial(
        pl.kernel,
        out_type=jax.ShapeDtypeStruct((NC, NPAD, D), jnp.float32),
        mesh=mesh,
        compiler_params=pltpu.CompilerParams(use_tc_tiling_on_sc=False),
        scratch_types=[
            pltpu.VMEM((NCHUNK, CHUNK), jnp.int32),
            pltpu.VMEM((NCHUNK, CHUNK), jnp.int32),
            pltpu.VMEM((CHUNK, D), jnp.float32),
            pltpu.VMEM_SHARED((NPAD, D), jnp.float32),
            pltpu.SemaphoreType.DMA,
        ],
    )
    def prop_kernel(y_hbm, src_hbm, dst_hbm, z_hbm, out_hbm, sidx, didx, gbuf,
                    acc, sem):
        cid = lax.axis_index("c")
        sid = lax.axis_index("s")
        wid = sid * NC + cid
        pltpu.sync_copy(src_hbm.at[wid], sidx)
        pltpu.sync_copy(dst_hbm.at[wid], didx)
        pltpu.sync_copy(z_hbm, acc.at[pl.ds(sid * RPT, RPT)])

        plsc.subcore_barrier()

        def body(j, carry):
            pltpu.async_copy(y_hbm.at[sidx.at[j]], gbuf, sem).wait()
            pltpu.sync_copy(gbuf, acc.at[didx.at[j]], add=True)
            return carry

        lax.fori_loop(0, NCHUNK, body, 0)
        plsc.subcore_barrier()
        pltpu.sync_copy(
            acc.at[pl.ds(sid * RPT, RPT)],
            out_hbm.at[cid, pl.ds(sid * RPT, RPT)],
        )

    return prop_kernel


_DEG = _make_sc_degree()
_PROP_HID = _make_sc_propagate(HID)
_PROP_16 = _make_sc_propagate(16)

BM = 1000


def _tc_pre(deg0, deg1, x, w1):
    def body(d0, d1, xr, wr, y1, dis16):
        deg = d0[:, 0:1] + d1[:, 0:1] + 1.0
        dis = lax.rsqrt(deg)
        xw = jnp.dot(xr[...], wr[...], preferred_element_type=jnp.float32)
        y1[...] = xw * dis
        dis16[...] = jnp.broadcast_to(dis, (BM, 16))

    return pl.pallas_call(
        body,
        grid=(N // BM,),
        in_specs=[
            pl.BlockSpec((BM, DEGW), lambda i: (i, 0)),
            pl.BlockSpec((BM, DEGW), lambda i: (i, 0)),
            pl.BlockSpec((BM, HID), lambda i: (i, 0)),
            pl.BlockSpec((HID, HID), lambda i: (0, 0)),
        ],
        out_specs=[
            pl.BlockSpec((BM, HID), lambda i: (i, 0)),
            pl.BlockSpec((BM, 16), lambda i: (i, 0)),
        ],
        out_shape=[
            jax.ShapeDtypeStruct((N, HID), jnp.float32),
            jax.ShapeDtypeStruct((N, 16), jnp.float32),
        ],
    )(deg0, deg1, x, w1)


def _tc_mid(s0, s1, y1, dis16, b1r, w2p):
    def body(a0, a1, yr, dr, br, wr, y2):
        dis = dr[:, 0:1]
        h = jnp.maximum(dis * (a0[...] + a1[...] + yr[...]) + br[...], 0.0)
        y2[...] = jnp.dot(h, wr[...], preferred_element_type=jnp.float32) * dis

    return pl.pallas_call(
        body,
        grid=(N // BM,),
        in_specs=[
            pl.BlockSpec((BM, HID), lambda i: (i, 0)),
            pl.BlockSpec((BM, HID), lambda i: (i, 0)),
            pl.BlockSpec((BM, HID), lambda i: (i, 0)),
            pl.BlockSpec((BM, 16), lambda i: (i, 0)),
            pl.BlockSpec((1, HID), lambda i: (0, 0)),
            pl.BlockSpec((HID, 16), lambda i: (0, 0)),
        ],
        out_specs=pl.BlockSpec((BM, 16), lambda i: (i, 0)),
        out_shape=jax.ShapeDtypeStruct((N, 16), jnp.float32),
    )(s0, s1, y1, dis16, b1r, w2p)


def _tc_post(t0, t1, y2, dis16, init, b2p, wh, wi, bf8):
    def body(a0, a1, yr, dr, ir, br, whr, wir, bfr, out8):
        dis = dr[:, 0:1]
        h2 = dis * (a0[...] + a1[...] + yr[...]) + br[...]
        out8[...] = (
            jnp.dot(h2, whr[...], preferred_element_type=jnp.float32)
            + jnp.dot(ir[...], wir[...], preferred_element_type=jnp.float32)
            + bfr[...]
        )

    return pl.pallas_call(
        body,
        grid=(N // BM,),
        in_specs=[
            pl.BlockSpec((BM, 16), lambda i: (i, 0)),
            pl.BlockSpec((BM, 16), lambda i: (i, 0)),
            pl.BlockSpec((BM, 16), lambda i: (i, 0)),
            pl.BlockSpec((BM, 16), lambda i: (i, 0)),
            pl.BlockSpec((BM, INIT_DIM), lambda i: (i, 0)),
            pl.BlockSpec((1, 16), lambda i: (0, 0)),
            pl.BlockSpec((16, 8), lambda i: (0, 0)),
            pl.BlockSpec((INIT_DIM, 8), lambda i: (0, 0)),
            pl.BlockSpec((1, 8), lambda i: (0, 0)),
        ],
        out_specs=pl.BlockSpec((BM, 8), lambda i: (i, 0)),
        out_shape=jax.ShapeDtypeStruct((N, 8), jnp.float32),
    )(t0, t1, y2, dis16, init, b2p, wh, wi, bf8)


def kernel(x, edge_index, edge_attr, initial_state, W1, b1, W2, b2, Wfc, bfc):
    del edge_attr
    pad_src = jnp.full((EP - E,), N, jnp.int32)
    pad_dst = jnp.full((EP - E,), NPAD - 1, jnp.int32)
    src4 = jnp.concatenate([edge_index[0], pad_src]).reshape(NW, NCHUNK, CHUNK)
    dst4 = jnp.concatenate([edge_index[1], pad_dst]).reshape(NW, NCHUNK, CHUNK)

    b1r = b1.reshape(1, HID)
    w2p = jnp.zeros((HID, 16), jnp.float32).at[:, :OUT].set(W2)
    b2p = jnp.zeros((1, 16), jnp.float32).at[0, :OUT].set(b2)
    wh = jnp.zeros((16, 8), jnp.float32).at[:OUT, :OUT].set(Wfc[:OUT])
    wi = jnp.zeros((INIT_DIM, 8), jnp.float32).at[:, :OUT].set(Wfc[OUT:])
    bf8 = jnp.zeros((1, 8), jnp.float32).at[0, :OUT].set(bfc)

    z16 = jnp.zeros((RPT, 16), jnp.float32)
    zhid = jnp.zeros((RPT, HID), jnp.float32)

    deg = _DEG(dst4, z16)
    y1, dis16 = _tc_pre(deg[0, :N], deg[1, :N], x, W1)
    y1p = jnp.pad(y1, ((0, NPAD - N), (0, 0)))
    s = _PROP_HID(y1p, src4, dst4, zhid)
    y2 = _tc_mid(s[0, :N], s[1, :N], y1, dis16, b1r, w2p)
    y2p = jnp.pad(y2, ((0, NPAD - N), (0, 0)))
    t = _PROP_16(y2p, src4, dst4, z16)
    out8 = _tc_post(t[0, :N], t[1, :N], y2, dis16, initial_state, b2p, wh, wi, bf8)
    return out8[:, :OUT]


# speedup vs baseline: 14.1439x; 1.0696x over previous
import functools

import jax
import jax.numpy as jnp
from jax import lax
from jax.experimental import pallas as pl
from jax.experimental.pallas import tpu as pltpu
from jax.experimental.pallas import tpu_sc as plsc

N = 10000
E = 320000
HID = 128
OUT = 3
INIT_DIM = 8

NC = 2
NS = 16
NW = NC * NS
CHUNK = 128
NCHUNK = 80
EPW = CHUNK * NCHUNK
EP = NW * EPW
NPAD = 10240
RPT = NPAD // NS
DEGW = 16


def _make_sc_degree():
    mesh = plsc.VectorSubcoreMesh(core_axis_name="c", subcore_axis_name="s")

    @functools.partial(
        pl.kernel,
        out_type=jax.ShapeDtypeStruct((NC, NPAD, DEGW), jnp.float32),
        mesh=mesh,
        compiler_params=pltpu.CompilerParams(use_tc_tiling_on_sc=False),
        scratch_types=[
            pltpu.VMEM((NCHUNK, CHUNK), jnp.int32),
            pltpu.VMEM((CHUNK, DEGW), jnp.float32),
            pltpu.VMEM_SHARED((NPAD, DEGW), jnp.float32),
        ],
    )
    def deg_kernel(dst_hbm, z_hbm, out_hbm, didx, ones, acc):
        cid = lax.axis_index("c")
        sid = lax.axis_index("s")
        wid = sid * NC + cid
        pltpu.sync_copy(dst_hbm.at[wid], didx)

        def fill_ones(i, carry):
            ones[i, :] = jnp.full((DEGW,), 1.0, jnp.float32)
            return carry

        lax.fori_loop(0, CHUNK, fill_ones, 0)
        pltpu.sync_copy(z_hbm, acc.at[pl.ds(sid * RPT, RPT)])
        plsc.subcore_barrier()

        def body(j, carry):
            pltpu.sync_copy(ones, acc.at[didx.at[j]], add=True)
            return carry

        lax.fori_loop(0, NCHUNK, body, 0)
        plsc.subcore_barrier()
        pltpu.sync_copy(
            acc.at[pl.ds(sid * RPT, RPT)],
            out_hbm.at[cid, pl.ds(sid * RPT, RPT)],
        )

    return deg_kernel


IB = 8
NBLK = NCHUNK // IB


def _make_sc_propagate(D):
    mesh = plsc.VectorSubcoreMesh(core_axis_name="c", subcore_axis_name="s")

    @functools.partial(
        pl.kernel,
        out_type=jax.ShapeDtypeStruct((NC, NPAD, D), jnp.float32),
        mesh=mesh,
        compiler_params=pltpu.CompilerParams(use_tc_tiling_on_sc=False),
        scratch_types=[
            pltpu.VMEM((2, IB, CHUNK), jnp.int32),
            pltpu.VMEM((2, IB, CHUNK), jnp.int32),
            pltpu.VMEM((2, CHUNK, D), jnp.float32),
            pltpu.VMEM_SHARED((NPAD, D), jnp.float32),
            pltpu.SemaphoreType.DMA,
            pltpu.SemaphoreType.DMA,
        ],
    )
    def prop_kernel(y_hbm, src_hbm, dst_hbm, z_hbm, out_hbm, sidx, didx, gbuf,
                    acc, gsem, isem):
        cid = lax.axis_index("c")
        sid = lax.axis_index("s")
        wid = sid * NC + cid
        pltpu.sync_copy(src_hbm.at[wid, pl.ds(0, IB)], sidx.at[0])
        pltpu.sync_copy(dst_hbm.at[wid, pl.ds(0, IB)], didx.at[0])
        pltpu.sync_copy(z_hbm, acc.at[pl.ds(sid * RPT, RPT)])

        pltpu.async_copy(y_hbm.at[sidx.at[0, 0]], gbuf.at[0], gsem)
        plsc.subcore_barrier()

        def blk(b, carry):
            p = b % 2

            @pl.when(b + 1 < NBLK)
            def _load_next_indices():
                pltpu.async_copy(src_hbm.at[wid, pl.ds((b + 1) * IB, IB)],
                                 sidx.at[(b + 1) % 2], isem)
                pltpu.async_copy(dst_hbm.at[wid, pl.ds((b + 1) * IB, IB)],
                                 didx.at[(b + 1) % 2], isem)

            def chunk(t, carry2):
                pltpu.make_async_copy(
                    y_hbm.at[sidx.at[p, t]], gbuf.at[t % 2], gsem).wait()

                @pl.when(t + 1 < IB)
                def _prefetch():
                    pltpu.async_copy(
                        y_hbm.at[sidx.at[p, t + 1]], gbuf.at[(t + 1) % 2], gsem)

                pltpu.sync_copy(gbuf.at[t % 2], acc.at[didx.at[p, t]], add=True)
                return carry2

            lax.fori_loop(0, IB, chunk, 0)

            @pl.when(b + 1 < NBLK)
            def _boundary_prefetch():
                pn = (b + 1) % 2
                pltpu.make_async_copy(src_hbm.at[wid, pl.ds((b + 1) * IB, IB)],
                                      sidx.at[pn], isem).wait()
                pltpu.make_async_copy(dst_hbm.at[wid, pl.ds((b + 1) * IB, IB)],
                                      didx.at[pn], isem).wait()
                pltpu.async_copy(y_hbm.at[sidx.at[pn, 0]], gbuf.at[0], gsem)

            return carry

        lax.fori_loop(0, NBLK, blk, 0)
        plsc.subcore_barrier()
        pltpu.sync_copy(
            acc.at[pl.ds(sid * RPT, RPT)],
            out_hbm.at[cid, pl.ds(sid * RPT, RPT)],
        )

    return prop_kernel


_DEG = _make_sc_degree()
_PROP_HID = _make_sc_propagate(HID)
_PROP_16 = _make_sc_propagate(16)

BM = 1000


def _tc_pre(deg0, deg1, x, w1):
    def body(d0, d1, xr, wr, y1, dis16):
        deg = d0[:, 0:1] + d1[:, 0:1] + 1.0
        dis = lax.rsqrt(deg)
        xw = jnp.dot(xr[...], wr[...], preferred_element_type=jnp.float32)
        y1[...] = xw * dis
        dis16[...] = jnp.broadcast_to(dis, (BM, 16))

    return pl.pallas_call(
        body,
        grid=(N // BM,),
        in_specs=[
            pl.BlockSpec((BM, DEGW), lambda i: (i, 0)),
            pl.BlockSpec((BM, DEGW), lambda i: (i, 0)),
            pl.BlockSpec((BM, HID), lambda i: (i, 0)),
            pl.BlockSpec((HID, HID), lambda i: (0, 0)),
        ],
        out_specs=[
            pl.BlockSpec((BM, HID), lambda i: (i, 0)),
            pl.BlockSpec((BM, 16), lambda i: (i, 0)),
        ],
        out_shape=[
            jax.ShapeDtypeStruct((N, HID), jnp.float32),
            jax.ShapeDtypeStruct((N, 16), jnp.float32),
        ],
    )(deg0, deg1, x, w1)


def _tc_mid(s0, s1, y1, dis16, b1r, w2p):
    def body(a0, a1, yr, dr, br, wr, y2):
        dis = dr[:, 0:1]
        h = jnp.maximum(dis * (a0[...] + a1[...] + yr[...]) + br[...], 0.0)
        y2[...] = jnp.dot(h, wr[...], preferred_element_type=jnp.float32) * dis

    return pl.pallas_call(
        body,
        grid=(N // BM,),
        in_specs=[
            pl.BlockSpec((BM, HID), lambda i: (i, 0)),
            pl.BlockSpec((BM, HID), lambda i: (i, 0)),
            pl.BlockSpec((BM, HID), lambda i: (i, 0)),
            pl.BlockSpec((BM, 16), lambda i: (i, 0)),
            pl.BlockSpec((1, HID), lambda i: (0, 0)),
            pl.BlockSpec((HID, 16), lambda i: (0, 0)),
        ],
        out_specs=pl.BlockSpec((BM, 16), lambda i: (i, 0)),
        out_shape=jax.ShapeDtypeStruct((N, 16), jnp.float32),
    )(s0, s1, y1, dis16, b1r, w2p)


def _tc_post(t0, t1, y2, dis16, init, b2p, wh, wi, bf8):
    def body(a0, a1, yr, dr, ir, br, whr, wir, bfr, out8):
        dis = dr[:, 0:1]
        h2 = dis * (a0[...] + a1[...] + yr[...]) + br[...]
        out8[...] = (
            jnp.dot(h2, whr[...], preferred_element_type=jnp.float32)
            + jnp.dot(ir[...], wir[...], preferred_element_type=jnp.float32)
            + bfr[...]
        )

    return pl.pallas_call(
        body,
        grid=(N // BM,),
        in_specs=[
            pl.BlockSpec((BM, 16), lambda i: (i, 0)),
            pl.BlockSpec((BM, 16), lambda i: (i, 0)),
            pl.BlockSpec((BM, 16), lambda i: (i, 0)),
            pl.BlockSpec((BM, 16), lambda i: (i, 0)),
            pl.BlockSpec((BM, INIT_DIM), lambda i: (i, 0)),
            pl.BlockSpec((1, 16), lambda i: (0, 0)),
            pl.BlockSpec((16, 8), lambda i: (0, 0)),
            pl.BlockSpec((INIT_DIM, 8), lambda i: (0, 0)),
            pl.BlockSpec((1, 8), lambda i: (0, 0)),
        ],
        out_specs=pl.BlockSpec((BM, 8), lambda i: (i, 0)),
        out_shape=jax.ShapeDtypeStruct((N, 8), jnp.float32),
    )(t0, t1, y2, dis16, init, b2p, wh, wi, bf8)


def kernel(x, edge_index, edge_attr, initial_state, W1, b1, W2, b2, Wfc, bfc):
    del edge_attr
    pad_src = jnp.full((EP - E,), N, jnp.int32)
    pad_dst = jnp.full((EP - E,), NPAD - 1, jnp.int32)
    src4 = jnp.concatenate([edge_index[0], pad_src]).reshape(NW, NCHUNK, CHUNK)
    dst4 = jnp.concatenate([edge_index[1], pad_dst]).reshape(NW, NCHUNK, CHUNK)

    b1r = b1.reshape(1, HID)
    w2p = jnp.zeros((HID, 16), jnp.float32).at[:, :OUT].set(W2)
    b2p = jnp.zeros((1, 16), jnp.float32).at[0, :OUT].set(b2)
    wh = jnp.zeros((16, 8), jnp.float32).at[:OUT, :OUT].set(Wfc[:OUT])
    wi = jnp.zeros((INIT_DIM, 8), jnp.float32).at[:, :OUT].set(Wfc[OUT:])
    bf8 = jnp.zeros((1, 8), jnp.float32).at[0, :OUT].set(bfc)

    z16 = jnp.zeros((RPT, 16), jnp.float32)
    zhid = jnp.zeros((RPT, HID), jnp.float32)

    deg = _DEG(dst4, z16)
    y1, dis16 = _tc_pre(deg[0, :N], deg[1, :N], x, W1)
    y1p = jnp.pad(y1, ((0, NPAD - N), (0, 0)))
    s = _PROP_HID(y1p, src4, dst4, zhid)
    y2 = _tc_mid(s[0, :N], s[1, :N], y1, dis16, b1r, w2p)
    y2p = jnp.pad(y2, ((0, NPAD - N), (0, 0)))
    t = _PROP_16(y2p, src4, dst4, z16)
    out8 = _tc_post(t[0, :N], t[1, :N], y2, dis16, initial_state, b2p, wh, wi, bf8)
    return out8[:, :OUT]


# speedup vs baseline: 26.0963x; 1.8451x over previous
import functools

import jax
import jax.numpy as jnp
from jax import lax
from jax.experimental import pallas as pl
from jax.experimental.pallas import tpu as pltpu
from jax.experimental.pallas import tpu_sc as plsc

N = 10000
E = 320000
HID = 128
OUT = 3
INIT_DIM = 8

NC = 2
NS = 16
NW = NC * NS
CHUNK = 128
NCHUNK = 80
EPW = CHUNK * NCHUNK
EP = NW * EPW
NPAD = 10240
RPT = NPAD // NS
DEGW = 16


def _make_sc_degree():
    mesh = plsc.VectorSubcoreMesh(core_axis_name="c", subcore_axis_name="s")

    @functools.partial(
        pl.kernel,
        out_type=jax.ShapeDtypeStruct((NC, NPAD, DEGW), jnp.float32),
        mesh=mesh,
        compiler_params=pltpu.CompilerParams(use_tc_tiling_on_sc=False),
        scratch_types=[
            pltpu.VMEM((NCHUNK, CHUNK), jnp.int32),
            pltpu.VMEM((CHUNK, DEGW), jnp.float32),
            pltpu.VMEM_SHARED((NPAD, DEGW), jnp.float32),
        ],
    )
    def deg_kernel(dst_hbm, z_hbm, out_hbm, didx, ones, acc):
        cid = lax.axis_index("c")
        sid = lax.axis_index("s")
        wid = sid * NC + cid
        pltpu.sync_copy(dst_hbm.at[wid], didx)

        def fill_ones(i, carry):
            ones[i, :] = jnp.full((DEGW,), 1.0, jnp.float32)
            return carry

        lax.fori_loop(0, CHUNK, fill_ones, 0)
        pltpu.sync_copy(z_hbm, acc.at[pl.ds(sid * RPT, RPT)])
        plsc.subcore_barrier()

        def body(j, carry):
            pltpu.sync_copy(ones, acc.at[didx.at[j]], add=True)
            return carry

        lax.fori_loop(0, NCHUNK, body, 0)
        plsc.subcore_barrier()
        pltpu.sync_copy(
            acc.at[pl.ds(sid * RPT, RPT)],
            out_hbm.at[cid, pl.ds(sid * RPT, RPT)],
        )

    return deg_kernel


IB = 8
NBLK = NCHUNK // IB


def _make_sc_propagate(D):
    mesh = plsc.VectorSubcoreMesh(core_axis_name="c", subcore_axis_name="s")

    @functools.partial(
        pl.kernel,
        out_type=jax.ShapeDtypeStruct((NC, NPAD, D), jnp.float32),
        mesh=mesh,
        compiler_params=pltpu.CompilerParams(use_tc_tiling_on_sc=False),
        scratch_types=[
            pltpu.VMEM((2, IB, CHUNK), jnp.int32),
            pltpu.VMEM((2, IB, CHUNK), jnp.int32),
            pltpu.VMEM((2, CHUNK, D), jnp.float32),
            pltpu.VMEM_SHARED((NPAD, D), jnp.float32),
            pltpu.VMEM_SHARED((NPAD, D), jnp.float32),
            pltpu.SemaphoreType.DMA,
            pltpu.SemaphoreType.DMA,
        ],
    )
    def prop_kernel(y_hbm, src_hbm, dst_hbm, z_hbm, out_hbm, sidx, didx, gbuf,
                    tbl, acc, gsem, isem):
        cid = lax.axis_index("c")
        sid = lax.axis_index("s")
        wid = sid * NC + cid
        pltpu.sync_copy(src_hbm.at[wid, pl.ds(0, IB)], sidx.at[0])
        pltpu.sync_copy(dst_hbm.at[wid, pl.ds(0, IB)], didx.at[0])
        pltpu.sync_copy(y_hbm.at[pl.ds(sid * RPT, RPT)],
                        tbl.at[pl.ds(sid * RPT, RPT)])
        pltpu.sync_copy(z_hbm, acc.at[pl.ds(sid * RPT, RPT)])
        plsc.subcore_barrier()

        pltpu.async_copy(tbl.at[sidx.at[0, 0]], gbuf.at[0], gsem)

        def blk(b, carry):
            p = b % 2

            @pl.when(b + 1 < NBLK)
            def _load_next_indices():
                pltpu.async_copy(src_hbm.at[wid, pl.ds((b + 1) * IB, IB)],
                                 sidx.at[(b + 1) % 2], isem)
                pltpu.async_copy(dst_hbm.at[wid, pl.ds((b + 1) * IB, IB)],
                                 didx.at[(b + 1) % 2], isem)

            def chunk(t, carry2):
                pltpu.make_async_copy(
                    tbl.at[sidx.at[p, t]], gbuf.at[t % 2], gsem).wait()

                @pl.when(t + 1 < IB)
                def _prefetch():
                    pltpu.async_copy(
                        tbl.at[sidx.at[p, t + 1]], gbuf.at[(t + 1) % 2], gsem)

                pltpu.sync_copy(gbuf.at[t % 2], acc.at[didx.at[p, t]], add=True)
                return carry2

            lax.fori_loop(0, IB, chunk, 0)

            @pl.when(b + 1 < NBLK)
            def _boundary_prefetch():
                pn = (b + 1) % 2
                pltpu.make_async_copy(src_hbm.at[wid, pl.ds((b + 1) * IB, IB)],
                                      sidx.at[pn], isem).wait()
                pltpu.make_async_copy(dst_hbm.at[wid, pl.ds((b + 1) * IB, IB)],
                                      didx.at[pn], isem).wait()
                pltpu.async_copy(tbl.at[sidx.at[pn, 0]], gbuf.at[0], gsem)

            return carry

        lax.fori_loop(0, NBLK, blk, 0)
        plsc.subcore_barrier()
        pltpu.sync_copy(
            acc.at[pl.ds(sid * RPT, RPT)],
            out_hbm.at[cid, pl.ds(sid * RPT, RPT)],
        )

    return prop_kernel


_DEG = _make_sc_degree()
_PROP_64 = _make_sc_propagate(64)
_PROP_16 = _make_sc_propagate(16)

BM = 1000


def _tc_pre(deg0, deg1, x, w1):
    def body(d0, d1, xr, wr, y1, dis16):
        deg = d0[:, 0:1] + d1[:, 0:1] + 1.0
        dis = lax.rsqrt(deg)
        xw = jnp.dot(xr[...], wr[...], preferred_element_type=jnp.float32)
        y1[...] = xw * dis
        dis16[...] = jnp.broadcast_to(dis, (BM, 16))

    return pl.pallas_call(
        body,
        grid=(N // BM,),
        in_specs=[
            pl.BlockSpec((BM, DEGW), lambda i: (i, 0)),
            pl.BlockSpec((BM, DEGW), lambda i: (i, 0)),
            pl.BlockSpec((BM, HID), lambda i: (i, 0)),
            pl.BlockSpec((HID, HID), lambda i: (0, 0)),
        ],
        out_specs=[
            pl.BlockSpec((BM, HID), lambda i: (i, 0)),
            pl.BlockSpec((BM, 16), lambda i: (i, 0)),
        ],
        out_shape=[
            jax.ShapeDtypeStruct((N, HID), jnp.float32),
            jax.ShapeDtypeStruct((N, 16), jnp.float32),
        ],
    )(deg0, deg1, x, w1)


def _tc_mid(s0lo, s1lo, s0hi, s1hi, y1, dis16, b1r, w2p):
    def body(a0l, a1l, a0h, a1h, yr, dr, br, wr, y2):
        dis = dr[:, 0:1]
        seg = jnp.concatenate([a0l[...] + a1l[...], a0h[...] + a1h[...]],
                              axis=1)
        h = jnp.maximum(dis * (seg + yr[...]) + br[...], 0.0)
        y2[...] = jnp.dot(h, wr[...], preferred_element_type=jnp.float32) * dis

    half = pl.BlockSpec((BM, HID // 2), lambda i: (i, 0))
    return pl.pallas_call(
        body,
        grid=(N // BM,),
        in_specs=[
            half, half, half, half,
            pl.BlockSpec((BM, HID), lambda i: (i, 0)),
            pl.BlockSpec((BM, 16), lambda i: (i, 0)),
            pl.BlockSpec((1, HID), lambda i: (0, 0)),
            pl.BlockSpec((HID, 16), lambda i: (0, 0)),
        ],
        out_specs=pl.BlockSpec((BM, 16), lambda i: (i, 0)),
        out_shape=jax.ShapeDtypeStruct((N, 16), jnp.float32),
    )(s0lo, s1lo, s0hi, s1hi, y1, dis16, b1r, w2p)


def _tc_post(t0, t1, y2, dis16, init, b2p, wh, wi, bf8):
    def body(a0, a1, yr, dr, ir, br, whr, wir, bfr, out8):
        dis = dr[:, 0:1]
        h2 = dis * (a0[...] + a1[...] + yr[...]) + br[...]
        out8[...] = (
            jnp.dot(h2, whr[...], preferred_element_type=jnp.float32)
            + jnp.dot(ir[...], wir[...], preferred_element_type=jnp.float32)
            + bfr[...]
        )

    return pl.pallas_call(
        body,
        grid=(N // BM,),
        in_specs=[
            pl.BlockSpec((BM, 16), lambda i: (i, 0)),
            pl.BlockSpec((BM, 16), lambda i: (i, 0)),
            pl.BlockSpec((BM, 16), lambda i: (i, 0)),
            pl.BlockSpec((BM, 16), lambda i: (i, 0)),
            pl.BlockSpec((BM, INIT_DIM), lambda i: (i, 0)),
            pl.BlockSpec((1, 16), lambda i: (0, 0)),
            pl.BlockSpec((16, 8), lambda i: (0, 0)),
            pl.BlockSpec((INIT_DIM, 8), lambda i: (0, 0)),
            pl.BlockSpec((1, 8), lambda i: (0, 0)),
        ],
        out_specs=pl.BlockSpec((BM, 8), lambda i: (i, 0)),
        out_shape=jax.ShapeDtypeStruct((N, 8), jnp.float32),
    )(t0, t1, y2, dis16, init, b2p, wh, wi, bf8)


def kernel(x, edge_index, edge_attr, initial_state, W1, b1, W2, b2, Wfc, bfc):
    del edge_attr
    pad_src = jnp.full((EP - E,), N, jnp.int32)
    pad_dst = jnp.full((EP - E,), NPAD - 1, jnp.int32)
    src4 = jnp.concatenate([edge_index[0], pad_src]).reshape(NW, NCHUNK, CHUNK)
    dst4 = jnp.concatenate([edge_index[1], pad_dst]).reshape(NW, NCHUNK, CHUNK)

    b1r = b1.reshape(1, HID)
    w2p = jnp.zeros((HID, 16), jnp.float32).at[:, :OUT].set(W2)
    b2p = jnp.zeros((1, 16), jnp.float32).at[0, :OUT].set(b2)
    wh = jnp.zeros((16, 8), jnp.float32).at[:OUT, :OUT].set(Wfc[:OUT])
    wi = jnp.zeros((INIT_DIM, 8), jnp.float32).at[:, :OUT].set(Wfc[OUT:])
    bf8 = jnp.zeros((1, 8), jnp.float32).at[0, :OUT].set(bfc)

    z16 = jnp.zeros((RPT, 16), jnp.float32)
    z64 = jnp.zeros((RPT, 64), jnp.float32)

    deg = _DEG(dst4, z16)
    y1, dis16 = _tc_pre(deg[0, :N], deg[1, :N], x, W1)
    y1p = jnp.pad(y1, ((0, NPAD - N), (0, 0)))
    slo = _PROP_64(y1p[:, :64], src4, dst4, z64)
    shi = _PROP_64(y1p[:, 64:], src4, dst4, z64)
    y2 = _tc_mid(slo[0, :N], slo[1, :N], shi[0, :N], shi[1, :N],
                 y1, dis16, b1r, w2p)
    y2p = jnp.pad(y2, ((0, NPAD - N), (0, 0)))
    t = _PROP_16(y2p, src4, dst4, z16)
    out8 = _tc_post(t[0, :N], t[1, :N], y2, dis16, initial_state, b2p, wh, wi, bf8)
    return out8[:, :OUT]


# speedup vs baseline: 27.9885x; 1.0725x over previous
import functools

import jax
import jax.numpy as jnp
from jax import lax
from jax.experimental import pallas as pl
from jax.experimental.pallas import tpu as pltpu
from jax.experimental.pallas import tpu_sc as plsc

N = 10000
E = 320000
HID = 128
OUT = 3
INIT_DIM = 8

NC = 2
NS = 16
NW = NC * NS
CHUNK = 128
NCHUNK = 80
EPW = CHUNK * NCHUNK
EP = NW * EPW
NPAD = 10240
RPT = NPAD // NS
DEGW = 16


def _make_sc_degree():
    mesh = plsc.VectorSubcoreMesh(core_axis_name="c", subcore_axis_name="s")

    @functools.partial(
        pl.kernel,
        out_type=jax.ShapeDtypeStruct((NC, NPAD, DEGW), jnp.float32),
        mesh=mesh,
        compiler_params=pltpu.CompilerParams(use_tc_tiling_on_sc=False),
        scratch_types=[
            pltpu.VMEM((NCHUNK, CHUNK), jnp.int32),
            pltpu.VMEM((CHUNK, DEGW), jnp.float32),
            pltpu.VMEM_SHARED((NPAD, DEGW), jnp.float32),
        ],
    )
    def deg_kernel(dst_hbm, z_hbm, out_hbm, didx, ones, acc):
        cid = lax.axis_index("c")
        sid = lax.axis_index("s")
        wid = sid * NC + cid
        pltpu.sync_copy(dst_hbm.at[wid], didx)

        def fill_ones(i, carry):
            ones[i, :] = jnp.full((DEGW,), 1.0, jnp.float32)
            return carry

        lax.fori_loop(0, CHUNK, fill_ones, 0)
        pltpu.sync_copy(z_hbm, acc.at[pl.ds(sid * RPT, RPT)])
        plsc.subcore_barrier()

        def body(j, carry):
            pltpu.sync_copy(ones, acc.at[didx.at[j]], add=True)
            return carry

        lax.fori_loop(0, NCHUNK, body, 0)
        plsc.subcore_barrier()
        pltpu.sync_copy(
            acc.at[pl.ds(sid * RPT, RPT)],
            out_hbm.at[cid, pl.ds(sid * RPT, RPT)],
        )

    return deg_kernel


IB = 8
NBLK = NCHUNK // IB


def _make_sc_propagate(D, NH):
    mesh = plsc.VectorSubcoreMesh(core_axis_name="c", subcore_axis_name="s")

    @functools.partial(
        pl.kernel,
        out_type=jax.ShapeDtypeStruct((NH, NC, NPAD, D), jnp.float32),
        mesh=mesh,
        compiler_params=pltpu.CompilerParams(use_tc_tiling_on_sc=False),
        scratch_types=[
            pltpu.VMEM((2, IB, CHUNK), jnp.int32),
            pltpu.VMEM((2, IB, CHUNK), jnp.int32),
            pltpu.VMEM((2, CHUNK, D), jnp.float32),
            pltpu.VMEM_SHARED((NPAD, D), jnp.float32),
            pltpu.VMEM_SHARED((NPAD, D), jnp.float32),
            pltpu.SemaphoreType.DMA,
            pltpu.SemaphoreType.DMA,
        ],
    )
    def prop_kernel(y_hbm, src_hbm, dst_hbm, z_hbm, out_hbm, sidx, didx, gbuf,
                    tbl, acc, gsem, isem):
        cid = lax.axis_index("c")
        sid = lax.axis_index("s")
        wid = sid * NC + cid

        def one_pass(h):
            pltpu.sync_copy(src_hbm.at[wid, pl.ds(0, IB)], sidx.at[0])
            pltpu.sync_copy(dst_hbm.at[wid, pl.ds(0, IB)], didx.at[0])
            pltpu.sync_copy(y_hbm.at[h, pl.ds(sid * RPT, RPT)],
                            tbl.at[pl.ds(sid * RPT, RPT)])
            pltpu.sync_copy(z_hbm, acc.at[pl.ds(sid * RPT, RPT)])
            plsc.subcore_barrier()

            pltpu.async_copy(tbl.at[sidx.at[0, 0]], gbuf.at[0], gsem)

            def blk(b, carry):
                p = b % 2

                @pl.when(b + 1 < NBLK)
                def _load_next_indices():
                    pltpu.async_copy(src_hbm.at[wid, pl.ds((b + 1) * IB, IB)],
                                     sidx.at[(b + 1) % 2], isem)
                    pltpu.async_copy(dst_hbm.at[wid, pl.ds((b + 1) * IB, IB)],
                                     didx.at[(b + 1) % 2], isem)

                def chunk(t, carry2):
                    pltpu.make_async_copy(
                        tbl.at[sidx.at[p, t]], gbuf.at[t % 2], gsem).wait()

                    @pl.when(t + 1 < IB)
                    def _prefetch():
                        pltpu.async_copy(
                            tbl.at[sidx.at[p, t + 1]], gbuf.at[(t + 1) % 2],
                            gsem)

                    pltpu.sync_copy(gbuf.at[t % 2], acc.at[didx.at[p, t]],
                                    add=True)
                    return carry2

                lax.fori_loop(0, IB, chunk, 0)

                @pl.when(b + 1 < NBLK)
                def _boundary_prefetch():
                    pn = (b + 1) % 2
                    pltpu.make_async_copy(
                        src_hbm.at[wid, pl.ds((b + 1) * IB, IB)],
                        sidx.at[pn], isem).wait()
                    pltpu.make_async_copy(
                        dst_hbm.at[wid, pl.ds((b + 1) * IB, IB)],
                        didx.at[pn], isem).wait()
                    pltpu.async_copy(tbl.at[sidx.at[pn, 0]], gbuf.at[0], gsem)

                return carry

            lax.fori_loop(0, NBLK, blk, 0)
            plsc.subcore_barrier()
            pltpu.sync_copy(
                acc.at[pl.ds(sid * RPT, RPT)],
                out_hbm.at[h, cid, pl.ds(sid * RPT, RPT)],
            )

        for h in range(NH):
            one_pass(h)

    return prop_kernel


_DEG = _make_sc_degree()
_PROP_64 = _make_sc_propagate(64, 2)
_PROP_16 = _make_sc_propagate(16, 1)

BM = 1000


def _tc_pre(deg, x, w1):
    def body(d0, d1, xr, wr, ys, dis16):
        dv = d0[0, :, 0:1] + d1[0, :, 0:1] + 1.0
        dis = lax.rsqrt(dv)
        xw = jnp.dot(xr[...], wr[...], preferred_element_type=jnp.float32)
        y = xw * dis
        ys[0] = y[:, :HID // 2]
        ys[1] = y[:, HID // 2:]
        dis16[...] = jnp.broadcast_to(dis, (BM, 16))

    return pl.pallas_call(
        body,
        grid=(N // BM,),
        in_specs=[
            pl.BlockSpec((1, BM, DEGW), lambda i: (0, i, 0)),
            pl.BlockSpec((1, BM, DEGW), lambda i: (1, i, 0)),
            pl.BlockSpec((BM, HID), lambda i: (i, 0)),
            pl.BlockSpec((HID, HID), lambda i: (0, 0)),
        ],
        out_specs=[
            pl.BlockSpec((2, BM, HID // 2), lambda i: (0, i, 0)),
            pl.BlockSpec((BM, 16), lambda i: (i, 0)),
        ],
        out_shape=[
            jax.ShapeDtypeStruct((2, NPAD, HID // 2), jnp.float32),
            jax.ShapeDtypeStruct((N, 16), jnp.float32),
        ],
    )(deg, deg, x, w1)


def _tc_mid(s, ys, dis16, b1r, w2p):
    def body(al0, al1, ah0, ah1, yl, yh, dr, br, wr, y2):
        dis = dr[:, 0:1]
        seg = jnp.concatenate(
            [al0[0, 0] + al1[0, 0] + yl[0], ah0[0, 0] + ah1[0, 0] + yh[0]],
            axis=1)
        h = jnp.maximum(dis * seg + br[...], 0.0)
        y2[0] = jnp.dot(h, wr[...], preferred_element_type=jnp.float32) * dis

    slab = lambda hh, cc: pl.BlockSpec((1, 1, BM, HID // 2),
                                       lambda i, hh=hh, cc=cc: (hh, cc, i, 0))
    yhalf = lambda hh: pl.BlockSpec((1, BM, HID // 2),
                                    lambda i, hh=hh: (hh, i, 0))
    return pl.pallas_call(
        body,
        grid=(N // BM,),
        in_specs=[
            slab(0, 0), slab(0, 1), slab(1, 0), slab(1, 1),
            yhalf(0), yhalf(1),
            pl.BlockSpec((BM, 16), lambda i: (i, 0)),
            pl.BlockSpec((1, HID), lambda i: (0, 0)),
            pl.BlockSpec((HID, 16), lambda i: (0, 0)),
        ],
        out_specs=pl.BlockSpec((1, BM, 16), lambda i: (0, i, 0)),
        out_shape=jax.ShapeDtypeStruct((1, NPAD, 16), jnp.float32),
    )(s, s, s, s, ys, ys, dis16, b1r, w2p)


def _tc_post(t, y2, dis16, init, b2p, wh, wi, bf8):
    def body(a0, a1, yr, dr, ir, br, whr, wir, bfr, out8):
        dis = dr[:, 0:1]
        h2 = dis * (a0[0, 0] + a1[0, 0] + yr[0]) + br[...]
        out8[...] = (
            jnp.dot(h2, whr[...], preferred_element_type=jnp.float32)
            + jnp.dot(ir[...], wir[...], preferred_element_type=jnp.float32)
            + bfr[...]
        )

    tslab = lambda cc: pl.BlockSpec((1, 1, BM, 16),
                                    lambda i, cc=cc: (0, cc, i, 0))
    return pl.pallas_call(
        body,
        grid=(N // BM,),
        in_specs=[
            tslab(0), tslab(1),
            pl.BlockSpec((1, BM, 16), lambda i: (0, i, 0)),
            pl.BlockSpec((BM, 16), lambda i: (i, 0)),
            pl.BlockSpec((BM, INIT_DIM), lambda i: (i, 0)),
            pl.BlockSpec((1, 16), lambda i: (0, 0)),
            pl.BlockSpec((16, 8), lambda i: (0, 0)),
            pl.BlockSpec((INIT_DIM, 8), lambda i: (0, 0)),
            pl.BlockSpec((1, 8), lambda i: (0, 0)),
        ],
        out_specs=pl.BlockSpec((BM, 8), lambda i: (i, 0)),
        out_shape=jax.ShapeDtypeStruct((N, 8), jnp.float32),
    )(t, t, y2, dis16, init, b2p, wh, wi, bf8)


def kernel(x, edge_index, edge_attr, initial_state, W1, b1, W2, b2, Wfc, bfc):
    del edge_attr
    pad_src = jnp.full((EP - E,), N, jnp.int32)
    pad_dst = jnp.full((EP - E,), NPAD - 1, jnp.int32)
    src4 = jnp.concatenate([edge_index[0], pad_src]).reshape(NW, NCHUNK, CHUNK)
    dst4 = jnp.concatenate([edge_index[1], pad_dst]).reshape(NW, NCHUNK, CHUNK)

    b1r = b1.reshape(1, HID)
    w2p = jnp.zeros((HID, 16), jnp.float32).at[:, :OUT].set(W2)
    b2p = jnp.zeros((1, 16), jnp.float32).at[0, :OUT].set(b2)
    wh = jnp.zeros((16, 8), jnp.float32).at[:OUT, :OUT].set(Wfc[:OUT])
    wi = jnp.zeros((INIT_DIM, 8), jnp.float32).at[:, :OUT].set(Wfc[OUT:])
    bf8 = jnp.zeros((1, 8), jnp.float32).at[0, :OUT].set(bfc)

    z16 = jnp.zeros((RPT, 16), jnp.float32)
    z64 = jnp.zeros((RPT, 64), jnp.float32)

    deg = _DEG(dst4, z16)
    ys, dis16 = _tc_pre(deg, x, W1)
    s = _PROP_64(ys, src4, dst4, z64)
    y2 = _tc_mid(s, ys, dis16, b1r, w2p)
    t = _PROP_16(y2, src4, dst4, z16)
    out8 = _tc_post(t, y2, dis16, initial_state, b2p, wh, wi, bf8)
    return out8[:, :OUT]


# speedup vs baseline: 35.5908x; 1.2716x over previous
import functools

import jax
import jax.numpy as jnp
from jax import lax
from jax.experimental import pallas as pl
from jax.experimental.pallas import tpu as pltpu
from jax.experimental.pallas import tpu_sc as plsc

N = 10000
E = 320000
HID = 128
OUT = 3
INIT_DIM = 8

NC = 2
NS = 16
NW = NC * NS
EPW = E // NW
NPAD = 10240
RPT = NPAD // NS
DEGW = 16
DCHUNK = 2000


def _make_sc_degree():
    mesh = plsc.VectorSubcoreMesh(core_axis_name="c", subcore_axis_name="s")

    @functools.partial(
        pl.kernel,
        out_type=jax.ShapeDtypeStruct((NC, NPAD, DEGW), jnp.float32),
        mesh=mesh,
        compiler_params=pltpu.CompilerParams(use_tc_tiling_on_sc=False),
        scratch_types=[
            pltpu.VMEM((EPW,), jnp.int32),
            pltpu.VMEM((DCHUNK, DEGW), jnp.float32),
            pltpu.VMEM_SHARED((NPAD, DEGW), jnp.float32),
        ],
    )
    def deg_kernel(ei_hbm, ones_hbm, z_hbm, out_hbm, didx, ones, acc):
        cid = lax.axis_index("c")
        sid = lax.axis_index("s")
        wid = sid * NC + cid
        pltpu.sync_copy(ei_hbm.at[1, pl.ds(wid * EPW, EPW)], didx)
        pltpu.sync_copy(ones_hbm, ones)
        pltpu.sync_copy(z_hbm, acc.at[pl.ds(sid * RPT, RPT)])
        plsc.subcore_barrier()

        def body(j, carry):
            pltpu.sync_copy(
                ones, acc.at[didx.at[pl.ds(j * DCHUNK, DCHUNK)]], add=True)
            return carry

        lax.fori_loop(0, EPW // DCHUNK, body, 0)
        plsc.subcore_barrier()
        pltpu.sync_copy(
            acc.at[pl.ds(sid * RPT, RPT)],
            out_hbm.at[cid, pl.ds(sid * RPT, RPT)],
        )

    return deg_kernel


def _make_sc_propagate(D, CHUNK, col_offs):
    NCHU = EPW // CHUNK
    mesh = plsc.VectorSubcoreMesh(core_axis_name="c", subcore_axis_name="s")

    @functools.partial(
        pl.kernel,
        out_type=jax.ShapeDtypeStruct((NC, NPAD, 128), jnp.float32),
        mesh=mesh,
        compiler_params=pltpu.CompilerParams(use_tc_tiling_on_sc=False),
        scratch_types=[
            pltpu.VMEM((EPW,), jnp.int32),
            pltpu.VMEM((EPW,), jnp.int32),
            pltpu.VMEM((2, CHUNK, D), jnp.float32),
            pltpu.VMEM_SHARED((NPAD, D), jnp.float32),
            pltpu.VMEM_SHARED((NPAD, D), jnp.float32),
            pltpu.SemaphoreType.DMA,
        ],
    )
    def prop_kernel(y_hbm, ei_hbm, z_hbm, out_hbm, sidx, didx, gbuf, tbl, acc,
                    gsem):
        cid = lax.axis_index("c")
        sid = lax.axis_index("s")
        wid = sid * NC + cid
        pltpu.sync_copy(ei_hbm.at[0, pl.ds(wid * EPW, EPW)], sidx)
        pltpu.sync_copy(ei_hbm.at[1, pl.ds(wid * EPW, EPW)], didx)

        def one_pass(c0):
            pltpu.sync_copy(y_hbm.at[pl.ds(sid * RPT, RPT), pl.ds(c0, D)],
                            tbl.at[pl.ds(sid * RPT, RPT)])
            pltpu.sync_copy(z_hbm, acc.at[pl.ds(sid * RPT, RPT)])
            plsc.subcore_barrier()

            pltpu.async_copy(tbl.at[sidx.at[pl.ds(0, CHUNK)]], gbuf.at[0],
                             gsem)

            def chunk(t, carry):
                pltpu.make_async_copy(
                    tbl.at[sidx.at[pl.ds(t * CHUNK, CHUNK)]],
                    gbuf.at[t % 2], gsem).wait()

                @pl.when(t + 1 < NCHU)
                def _prefetch():
                    pltpu.async_copy(
                        tbl.at[sidx.at[pl.ds((t + 1) * CHUNK, CHUNK)]],
                        gbuf.at[(t + 1) % 2], gsem)

                pltpu.sync_copy(
                    gbuf.at[t % 2],
                    acc.at[didx.at[pl.ds(t * CHUNK, CHUNK)]], add=True)
                return carry

            lax.fori_loop(0, NCHU, chunk, 0)
            plsc.subcore_barrier()
            pltpu.sync_copy(
                acc.at[pl.ds(sid * RPT, RPT)],
                out_hbm.at[cid, pl.ds(sid * RPT, RPT), pl.ds(c0, D)],
            )

        for c0 in col_offs:
            one_pass(c0)

    return prop_kernel


_DEG = _make_sc_degree()
_PROP_64 = _make_sc_propagate(64, 200, (0, 64))
_PROP_16 = _make_sc_propagate(16, 1000, (0,))

BM = 1000


def _tc_pre(deg, x, w1):
    def body(d0, d1, xr, wr, y_out, dis16):
        dv = d0[0, :, 0:1] + d1[0, :, 0:1] + 1.0
        dis = lax.rsqrt(dv)
        xw = jnp.dot(xr[...], wr[...], preferred_element_type=jnp.float32)
        y_out[...] = xw * dis
        dis16[...] = jnp.broadcast_to(dis, (BM, 16))

    return pl.pallas_call(
        body,
        grid=(N // BM,),
        in_specs=[
            pl.BlockSpec((1, BM, DEGW), lambda i: (0, i, 0)),
            pl.BlockSpec((1, BM, DEGW), lambda i: (1, i, 0)),
            pl.BlockSpec((BM, HID), lambda i: (i, 0)),
            pl.BlockSpec((HID, HID), lambda i: (0, 0)),
        ],
        out_specs=[
            pl.BlockSpec((BM, HID), lambda i: (i, 0)),
            pl.BlockSpec((BM, 16), lambda i: (i, 0)),
        ],
        out_shape=[
            jax.ShapeDtypeStruct((NPAD, HID), jnp.float32),
            jax.ShapeDtypeStruct((N, 16), jnp.float32),
        ],
    )(deg, deg, x, w1)


def _tc_mid(s, ys, dis16, b1r, w2p):
    def body(a0, a1, yr, dr, br, wr, y2):
        dis = dr[:, 0:1]
        seg = a0[0] + a1[0] + yr[...]
        h = jnp.maximum(dis * seg + br[...], 0.0)
        y2[...] = jnp.dot(h, wr[...], preferred_element_type=jnp.float32) * dis

    slab = lambda cc: pl.BlockSpec((1, BM, HID), lambda i, cc=cc: (cc, i, 0))
    return pl.pallas_call(
        body,
        grid=(N // BM,),
        in_specs=[
            slab(0), slab(1),
            pl.BlockSpec((BM, HID), lambda i: (i, 0)),
            pl.BlockSpec((BM, 16), lambda i: (i, 0)),
            pl.BlockSpec((1, HID), lambda i: (0, 0)),
            pl.BlockSpec((HID, HID), lambda i: (0, 0)),
        ],
        out_specs=pl.BlockSpec((BM, HID), lambda i: (i, 0)),
        out_shape=jax.ShapeDtypeStruct((NPAD, HID), jnp.float32),
    )(s, s, ys, dis16, b1r, w2p)


def _tc_post(t, y2, dis16, init, b2p, wh, wi, bf8):
    def body(a0, a1, yr, dr, ir, br, whr, wir, bfr, out8):
        dis = dr[:, 0:1]
        h2 = dis * (a0[0, :, :16] + a1[0, :, :16] + yr[:, :16]) + br[...]
        out8[...] = (
            jnp.dot(h2, whr[...], preferred_element_type=jnp.float32)
            + jnp.dot(ir[...], wir[...], preferred_element_type=jnp.float32)
            + bfr[...]
        )

    tslab = lambda cc: pl.BlockSpec((1, BM, 128), lambda i, cc=cc: (cc, i, 0))
    return pl.pallas_call(
        body,
        grid=(N // BM,),
        in_specs=[
            tslab(0), tslab(1),
            pl.BlockSpec((BM, 128), lambda i: (i, 0)),
            pl.BlockSpec((BM, 16), lambda i: (i, 0)),
            pl.BlockSpec((BM, INIT_DIM), lambda i: (i, 0)),
            pl.BlockSpec((1, 16), lambda i: (0, 0)),
            pl.BlockSpec((16, 8), lambda i: (0, 0)),
            pl.BlockSpec((INIT_DIM, 8), lambda i: (0, 0)),
            pl.BlockSpec((1, 8), lambda i: (0, 0)),
        ],
        out_specs=pl.BlockSpec((BM, 8), lambda i: (i, 0)),
        out_shape=jax.ShapeDtypeStruct((N, 8), jnp.float32),
    )(t, t, y2, dis16, init, b2p, wh, wi, bf8)


def kernel(x, edge_index, edge_attr, initial_state, W1, b1, W2, b2, Wfc, bfc):
    del edge_attr
    ei = edge_index.astype(jnp.int32)

    b1r = b1.reshape(1, HID)
    w2p = jnp.zeros((HID, HID), jnp.float32).at[:, :OUT].set(W2)
    b2p = jnp.zeros((1, 16), jnp.float32).at[0, :OUT].set(b2)
    wh = jnp.zeros((16, 8), jnp.float32).at[:OUT, :OUT].set(Wfc[:OUT])
    wi = jnp.zeros((INIT_DIM, 8), jnp.float32).at[:, :OUT].set(Wfc[OUT:])
    bf8 = jnp.zeros((1, 8), jnp.float32).at[0, :OUT].set(bfc)

    ones_deg = jnp.ones((DCHUNK, DEGW), jnp.float32)
    z16 = jnp.zeros((RPT, 16), jnp.float32)
    z64 = jnp.zeros((RPT, 64), jnp.float32)

    deg = _DEG(ei, ones_deg, z16)
    ys, dis16 = _tc_pre(deg, x, W1)
    s = _PROP_64(ys, ei, z64)
    y2 = _tc_mid(s, ys, dis16, b1r, w2p)
    t = _PROP_16(y2, ei, z16)
    out8 = _tc_post(t, y2, dis16, initial_state, b2p, wh, wi, bf8)
    return out8[:, :OUT]


# speedup vs baseline: 36.3229x; 1.0206x over previous
import functools

import jax
import jax.numpy as jnp
from jax import lax
from jax.experimental import pallas as pl
from jax.experimental.pallas import tpu as pltpu
from jax.experimental.pallas import tpu_sc as plsc

N = 10000
E = 320000
HID = 128
OUT = 3
INIT_DIM = 8

NC = 2
NS = 16
NW = NC * NS
EPW = E // NW
NPAD = 10240
RPT = NPAD // NS
DEGW = 16
DCHUNK = 2000


def _make_sc_degree():
    mesh = plsc.VectorSubcoreMesh(core_axis_name="c", subcore_axis_name="s")

    @functools.partial(
        pl.kernel,
        out_type=jax.ShapeDtypeStruct((NC, NPAD, DEGW), jnp.float32),
        mesh=mesh,
        compiler_params=pltpu.CompilerParams(use_tc_tiling_on_sc=False),
        scratch_types=[
            pltpu.VMEM((EPW,), jnp.int32),
            pltpu.VMEM((DCHUNK, DEGW), jnp.float32),
            pltpu.VMEM_SHARED((NPAD, DEGW), jnp.float32),
        ],
    )
    def deg_kernel(ei_hbm, ones_hbm, z_hbm, out_hbm, didx, ones, acc):
        cid = lax.axis_index("c")
        sid = lax.axis_index("s")
        wid = sid * NC + cid
        pltpu.sync_copy(ei_hbm.at[1, pl.ds(wid * EPW, EPW)], didx)
        pltpu.sync_copy(ones_hbm, ones)
        pltpu.sync_copy(z_hbm, acc.at[pl.ds(sid * RPT, RPT)])
        plsc.subcore_barrier()

        def body(j, carry):
            pltpu.sync_copy(
                ones, acc.at[didx.at[pl.ds(j * DCHUNK, DCHUNK)]], add=True)
            return carry

        lax.fori_loop(0, EPW // DCHUNK, body, 0)
        plsc.subcore_barrier()
        pltpu.sync_copy(
            acc.at[pl.ds(sid * RPT, RPT)],
            out_hbm.at[cid, pl.ds(sid * RPT, RPT)],
        )

    return deg_kernel


def _make_sc_propagate(D, CHUNK, col_offs):
    NCHU = EPW // CHUNK
    mesh = plsc.VectorSubcoreMesh(core_axis_name="c", subcore_axis_name="s")

    @functools.partial(
        pl.kernel,
        out_type=jax.ShapeDtypeStruct((NC, NPAD, 128), jnp.float32),
        mesh=mesh,
        compiler_params=pltpu.CompilerParams(use_tc_tiling_on_sc=False),
        scratch_types=[
            pltpu.VMEM((EPW,), jnp.int32),
            pltpu.VMEM((EPW,), jnp.int32),
            pltpu.VMEM((2, CHUNK, D), jnp.float32),
            pltpu.VMEM_SHARED((NPAD, D), jnp.float32),
            pltpu.VMEM_SHARED((NPAD, D), jnp.float32),
            pltpu.SemaphoreType.DMA,
        ],
    )
    def prop_kernel(y_hbm, ei_hbm, z_hbm, out_hbm, sidx, didx, gbuf, tbl, acc,
                    gsem):
        cid = lax.axis_index("c")
        sid = lax.axis_index("s")
        wid = sid * NC + cid
        pltpu.sync_copy(ei_hbm.at[0, pl.ds(wid * EPW, EPW)], sidx)
        pltpu.sync_copy(ei_hbm.at[1, pl.ds(wid * EPW, EPW)], didx)

        def one_pass(c0):
            pltpu.sync_copy(y_hbm.at[pl.ds(sid * RPT, RPT), pl.ds(c0, D)],
                            tbl.at[pl.ds(sid * RPT, RPT)])
            pltpu.sync_copy(z_hbm, acc.at[pl.ds(sid * RPT, RPT)])
            plsc.subcore_barrier()

            pltpu.async_copy(tbl.at[sidx.at[pl.ds(0, CHUNK)]], gbuf.at[0],
                             gsem)

            def chunk(t, carry):
                pltpu.make_async_copy(
                    tbl.at[sidx.at[pl.ds(t * CHUNK, CHUNK)]],
                    gbuf.at[t % 2], gsem).wait()

                @pl.when(t + 1 < NCHU)
                def _prefetch():
                    pltpu.async_copy(
                        tbl.at[sidx.at[pl.ds((t + 1) * CHUNK, CHUNK)]],
                        gbuf.at[(t + 1) % 2], gsem)

                pltpu.sync_copy(
                    gbuf.at[t % 2],
                    acc.at[didx.at[pl.ds(t * CHUNK, CHUNK)]], add=True)
                return carry

            lax.fori_loop(0, NCHU, chunk, 0)
            plsc.subcore_barrier()
            pltpu.sync_copy(
                acc.at[pl.ds(sid * RPT, RPT)],
                out_hbm.at[cid, pl.ds(sid * RPT, RPT), pl.ds(c0, D)],
            )

        for c0 in col_offs:
            one_pass(c0)

    return prop_kernel


_DEG = _make_sc_degree()
_PROP_64 = _make_sc_propagate(64, 200, (0, 64))
_PROP_16 = _make_sc_propagate(16, 1000, (0,))

BM = 2000


def _tc_mm(x, w1):
    def body(xr, wr, xw):
        xw[...] = jnp.dot(xr[...], wr[...], preferred_element_type=jnp.float32)

    return pl.pallas_call(
        body,
        grid=(N // BM,),
        in_specs=[
            pl.BlockSpec((BM, HID), lambda i: (i, 0)),
            pl.BlockSpec((HID, HID), lambda i: (0, 0)),
        ],
        out_specs=pl.BlockSpec((BM, HID), lambda i: (i, 0)),
        out_shape=jax.ShapeDtypeStruct((N, HID), jnp.float32),
    )(x, w1)


def _tc_scale(deg, xw):
    def body(d0, d1, xwr, y_out, dis16):
        dv = d0[0, :, 0:1] + d1[0, :, 0:1] + 1.0
        dis = lax.rsqrt(dv)
        y_out[...] = xwr[...] * dis
        dis16[...] = jnp.broadcast_to(dis, (BM, 16))

    return pl.pallas_call(
        body,
        grid=(N // BM,),
        in_specs=[
            pl.BlockSpec((1, BM, DEGW), lambda i: (0, i, 0)),
            pl.BlockSpec((1, BM, DEGW), lambda i: (1, i, 0)),
            pl.BlockSpec((BM, HID), lambda i: (i, 0)),
        ],
        out_specs=[
            pl.BlockSpec((BM, HID), lambda i: (i, 0)),
            pl.BlockSpec((BM, 16), lambda i: (i, 0)),
        ],
        out_shape=[
            jax.ShapeDtypeStruct((NPAD, HID), jnp.float32),
            jax.ShapeDtypeStruct((N, 16), jnp.float32),
        ],
    )(deg, deg, xw)


def _tc_mid(s, ys, dis16, b1r, w2p):
    def body(a0, a1, yr, dr, br, wr, y2):
        dis = dr[:, 0:1]
        seg = a0[0] + a1[0] + yr[...]
        h = jnp.maximum(dis * seg + br[...], 0.0)
        y2[...] = jnp.dot(h, wr[...], preferred_element_type=jnp.float32) * dis

    slab = lambda cc: pl.BlockSpec((1, BM, HID), lambda i, cc=cc: (cc, i, 0))
    return pl.pallas_call(
        body,
        grid=(N // BM,),
        in_specs=[
            slab(0), slab(1),
            pl.BlockSpec((BM, HID), lambda i: (i, 0)),
            pl.BlockSpec((BM, 16), lambda i: (i, 0)),
            pl.BlockSpec((1, HID), lambda i: (0, 0)),
            pl.BlockSpec((HID, HID), lambda i: (0, 0)),
        ],
        out_specs=pl.BlockSpec((BM, HID), lambda i: (i, 0)),
        out_shape=jax.ShapeDtypeStruct((NPAD, HID), jnp.float32),
    )(s, s, ys, dis16, b1r, w2p)


def _tc_post(t, y2, dis16, init, b2p, wh, wi, bf8):
    def body(a0, a1, yr, dr, ir, br, whr, wir, bfr, out8):
        dis = dr[:, 0:1]
        h2 = dis * (a0[0, :, :16] + a1[0, :, :16] + yr[:, :16]) + br[...]
        out8[...] = (
            jnp.dot(h2, whr[...], preferred_element_type=jnp.float32)
            + jnp.dot(ir[...], wir[...], preferred_element_type=jnp.float32)
            + bfr[...]
        )

    tslab = lambda cc: pl.BlockSpec((1, BM, 128), lambda i, cc=cc: (cc, i, 0))
    return pl.pallas_call(
        body,
        grid=(N // BM,),
        in_specs=[
            tslab(0), tslab(1),
            pl.BlockSpec((BM, 128), lambda i: (i, 0)),
            pl.BlockSpec((BM, 16), lambda i: (i, 0)),
            pl.BlockSpec((BM, INIT_DIM), lambda i: (i, 0)),
            pl.BlockSpec((1, 16), lambda i: (0, 0)),
            pl.BlockSpec((16, 8), lambda i: (0, 0)),
            pl.BlockSpec((INIT_DIM, 8), lambda i: (0, 0)),
            pl.BlockSpec((1, 8), lambda i: (0, 0)),
        ],
        out_specs=pl.BlockSpec((BM, 8), lambda i: (i, 0)),
        out_shape=jax.ShapeDtypeStruct((N, 8), jnp.float32),
    )(t, t, y2, dis16, init, b2p, wh, wi, bf8)


def kernel(x, edge_index, edge_attr, initial_state, W1, b1, W2, b2, Wfc, bfc):
    del edge_attr
    ei = edge_index.astype(jnp.int32)

    b1r = b1.reshape(1, HID)
    w2p = jnp.zeros((HID, HID), jnp.float32).at[:, :OUT].set(W2)
    b2p = jnp.zeros((1, 16), jnp.float32).at[0, :OUT].set(b2)
    wh = jnp.zeros((16, 8), jnp.float32).at[:OUT, :OUT].set(Wfc[:OUT])
    wi = jnp.zeros((INIT_DIM, 8), jnp.float32).at[:, :OUT].set(Wfc[OUT:])
    bf8 = jnp.zeros((1, 8), jnp.float32).at[0, :OUT].set(bfc)

    ones_deg = jnp.ones((DCHUNK, DEGW), jnp.float32)
    z16 = jnp.zeros((RPT, 16), jnp.float32)
    z64 = jnp.zeros((RPT, 64), jnp.float32)

    xw = _tc_mm(x, W1)
    deg = _DEG(ei, ones_deg, z16)
    ys, dis16 = _tc_scale(deg, xw)
    s = _PROP_64(ys, ei, z64)
    y2 = _tc_mid(s, ys, dis16, b1r, w2p)
    t = _PROP_16(y2, ei, z16)
    out8 = _tc_post(t, y2, dis16, initial_state, b2p, wh, wi, bf8)
    return out8[:, :OUT]
